# baseline (speedup 1.0000x reference)
import functools

import jax
import jax.numpy as jnp
from jax import lax
from jax.experimental import pallas as pl
from jax.experimental.pallas import tpu as pltpu
from jax.experimental.pallas import tpu_sc as plsc

B = 16384
D = 64
V = 100000
L = 16
NC = 2
NS = 16
NW = NC * NS
CPW = D // NW
CHUNK = 4096
NCH = B // CHUNK
GR = CHUNK // L


def _make_sc_kernel():
    mesh = plsc.VectorSubcoreMesh(core_axis_name="c", subcore_axis_name="s")

    @functools.partial(
        pl.kernel,
        mesh=mesh,
        out_type=jax.ShapeDtypeStruct((D, B), jnp.float32),
        compiler_params=pltpu.CompilerParams(needs_layout_passes=False,
                                             use_tc_tiling_on_sc=True),
        scratch_types=[
            pltpu.VMEM((V,), jnp.float32),
            pltpu.VMEM((B,), jnp.float32),
            pltpu.VMEM((2, CHUNK), jnp.int32),
            pltpu.SemaphoreType.DMA,
            pltpu.SemaphoreType.DMA,
            pltpu.SemaphoreType.DMA,
            pltpu.SemaphoreType.DMA,
        ],
    )
    def sc_body(ut_hbm, it_hbm, uix_hbm, iix_hbm, out_hbm,
                col_v, val_v, ixc_v, semc, semi0, semi1, semo):
        wid = lax.axis_index("s") * NC + lax.axis_index("c")
        semi = (semi0, semi1)
        out_cp = None

        for r in range(CPW):
            c = wid * CPW + r

            for tbl in range(2):
                table = ut_hbm if tbl == 0 else it_hbm
                ix_hbm = uix_hbm if tbl == 0 else iix_hbm
                ccp = pltpu.async_copy(table.at[c], col_v, semc)
                cps = {0: pltpu.async_copy(ix_hbm.at[pl.ds(0, CHUNK)],
                                           ixc_v.at[0], semi[0])}
                ccp.wait()
                if tbl == 0 and out_cp is not None:
                    out_cp.wait()
                for k in range(NCH):
                    if k + 1 < NCH:
                        nb = (k + 1) % 2
                        cps[k + 1] = pltpu.async_copy(
                            ix_hbm.at[pl.ds((k + 1) * CHUNK, CHUNK)],
                            ixc_v.at[nb], semi[nb])
                    cps[k].wait()
                    base = k * CHUNK
                    buf = k % 2

                    if tbl == 0:
                        @plsc.parallel_loop(0, GR, unroll=4)
                        def ubody(g, base=base, buf=buf):
                            off = pl.multiple_of(g * L, L)
                            ix = ixc_v[buf, pl.ds(off, L)]
                            val_v[pl.ds(base + off, L)] = (
                                plsc.load_gather(col_v, [ix]))
                    else:
                        @plsc.parallel_loop(0, GR, unroll=4)
                        def ibody(g, base=base, buf=buf):
                            off = pl.multiple_of(g * L, L)
                            pos = base + off
                            ix = ixc_v[buf, pl.ds(off, L)]
                            iv = plsc.load_gather(col_v, [ix])
                            val_v[pl.ds(pos, L)] = val_v[pl.ds(pos, L)] * iv

            out_cp = pltpu.async_copy(val_v, out_hbm.at[c], semo)

        out_cp.wait()

    return sc_body


_sc_kernel = _make_sc_kernel()


def kernel(inputs, user_embedding, item_embedding):
    prods = _sc_kernel(user_embedding.T, item_embedding.T,
                       inputs[:, 0], inputs[:, 1])
    return jnp.sum(prods, axis=0)
